# baseline (speedup 1.0000x reference)
import jax
import jax.numpy as jnp
from jax import lax
from jax.experimental import pallas as pl
from jax.experimental.pallas import tpu as pltpu
from jax.experimental.pallas import tpu_sc as plsc

D = 64
B0, B1 = 4096, 200
NW = 32
PER_W = B0 // NW * B1
UNIT = 256
N_UNITS = PER_W // UNIT
NSLOT = 4

_mesh = plsc.VectorSubcoreMesh(core_axis_name="core", subcore_axis_name="subcore")


def _flatten_idx(xT):
    @pl.kernel(
        out_type=jax.ShapeDtypeStruct((B0 * B1,), jnp.int32),
        mesh=_mesh,
        compiler_params=pltpu.CompilerParams(needs_layout_passes=False),
        scratch_types=[
            pltpu.VMEM((B1, 128), jnp.int32),
            pltpu.VMEM((PER_W,), jnp.int32),
        ],
    )
    def k(xt, xfl, xtb, xfb):
        w = lax.axis_index("subcore") * 2 + lax.axis_index("core")
        iota = lax.iota(jnp.int32, 16)
        i0 = w * 128
        pltpu.sync_copy(xt.at[:, pl.ds(i0, 128)], xtb)
        j_chunks = list(range(0, B1 - 16, 16)) + [B1 - 16]

        @pl.loop(0, 128)
        def _(ii):
            ci = iota * 0 + ii
            for j0 in j_chunks:
                v = plsc.load_gather(xtb, [iota + j0, ci])
                xfb[pl.ds(ii * B1 + j0, 16)] = v

        pltpu.sync_copy(xfb, xfl.at[pl.ds(w * PER_W, PER_W)])

    return k(xT)


def _gather(table, xfl):
    @pl.kernel(
        out_type=jax.ShapeDtypeStruct((B0 * B1, D), jnp.float32),
        mesh=_mesh,
        compiler_params=pltpu.CompilerParams(
            use_tc_tiling_on_sc=False, needs_layout_passes=False),
        scratch_types=[
            pltpu.VMEM((PER_W,), jnp.int32),
            pltpu.VMEM((UNIT, D), jnp.float32),
            pltpu.VMEM((UNIT, D), jnp.float32),
            pltpu.VMEM((UNIT, D), jnp.float32),
            pltpu.VMEM((UNIT, D), jnp.float32),
            pltpu.SemaphoreType.DMA,
            pltpu.SemaphoreType.DMA,
            pltpu.SemaphoreType.DMA,
            pltpu.SemaphoreType.DMA,
            pltpu.SemaphoreType.DMA,
            pltpu.SemaphoreType.DMA,
            pltpu.SemaphoreType.DMA,
            pltpu.SemaphoreType.DMA,
        ],
    )
    def k(tab, xf, out, xfb, g0, g1, g2, g3,
          sg0, sg1, sg2, sg3, so0, so1, so2, so3):
        w = lax.axis_index("subcore") * 2 + lax.axis_index("core")
        gbufs = (g0, g1, g2, g3)
        sgs = (sg0, sg1, sg2, sg3)
        sos = (so0, so1, so2, so3)
        base = w * PER_W

        pltpu.sync_copy(xf.at[pl.ds(base, PER_W)], xfb)

        def gather_copy(u, b):
            return pltpu.make_async_copy(
                tab.at[xfb.at[pl.ds(u * UNIT, UNIT)]], gbufs[b], sgs[b])

        def write_copy(u, b):
            return pltpu.make_async_copy(
                gbufs[b], out.at[pl.ds(base + u * UNIT, UNIT), :], sos[b])

        gather_copy(0, 0).start()
        gather_copy(1, 1).start()

        @pl.loop(0, N_UNITS, step=NSLOT)
        def _(u0):
            for db in range(NSLOT):
                u = u0 + db
                b = db % NSLOT
                nb = (db + 2) % NSLOT

                @pl.when(u >= 2)
                def _():
                    write_copy(u - 2, nb).wait()

                @pl.when(u + 2 < N_UNITS)
                def _():
                    gather_copy(u + 2, nb).start()

                gather_copy(u, b).wait()
                write_copy(u, b).start()

        write_copy(N_UNITS - 2, (N_UNITS - 2) % NSLOT).wait()
        write_copy(N_UNITS - 1, (N_UNITS - 1) % NSLOT).wait()

    return k(table, xfl)


def kernel(x, table):
    xT = x.T.astype(jnp.int32)
    xfl = _flatten_idx(xT)
    xfl, table = lax.optimization_barrier((xfl, table))
    out = _gather(table, xfl)
    return out.reshape(B0, B1, D)


# speedup vs baseline: 1.1165x; 1.1165x over previous
import jax
import jax.numpy as jnp
from jax import lax
from jax.experimental import pallas as pl
from jax.experimental.pallas import tpu as pltpu
from jax.experimental.pallas import tpu_sc as plsc

D = 64
B0, B1 = 4096, 200
NW = 32
PER_W = B0 // NW * B1
UNIT = 256
N_UNITS = PER_W // UNIT
NSLOT = 4

_mesh = plsc.VectorSubcoreMesh(core_axis_name="core", subcore_axis_name="subcore")


def _flatten_idx(xT):
    @pl.kernel(
        out_type=jax.ShapeDtypeStruct((B0 * B1,), jnp.int32),
        mesh=_mesh,
        compiler_params=pltpu.CompilerParams(needs_layout_passes=False),
        scratch_types=[
            pltpu.VMEM((B1, 128), jnp.int32),
            pltpu.VMEM((PER_W,), jnp.int32),
        ],
    )
    def k(xt, xfl, xtb, xfb):
        w = lax.axis_index("subcore") * 2 + lax.axis_index("core")
        iota = lax.iota(jnp.int32, 16)
        i0 = w * 128
        pltpu.sync_copy(xt.at[:, pl.ds(i0, 128)], xtb)
        j_chunks = list(range(0, B1 - 16, 16)) + [B1 - 16]

        @pl.loop(0, 128)
        def _(ii):
            ci = iota * 0 + ii
            for j0 in j_chunks:
                v = plsc.load_gather(xtb, [iota + j0, ci])
                xfb[pl.ds(ii * B1 + j0, 16)] = v

        pltpu.sync_copy(xfb, xfl.at[pl.ds(w * PER_W, PER_W)])

    return k(xT)


def _gather(table, xfl):
    @pl.kernel(
        out_type=jax.ShapeDtypeStruct((B0 * B1, D), jnp.float32),
        mesh=_mesh,
        compiler_params=pltpu.CompilerParams(
            use_tc_tiling_on_sc=False, needs_layout_passes=False),
        scratch_types=[
            pltpu.VMEM((PER_W,), jnp.int32),
            pltpu.VMEM((UNIT, D), jnp.float32),
            pltpu.VMEM((UNIT, D), jnp.float32),
            pltpu.VMEM((UNIT, D), jnp.float32),
            pltpu.VMEM((UNIT, D), jnp.float32),
            pltpu.SemaphoreType.DMA,
            pltpu.SemaphoreType.DMA,
            pltpu.SemaphoreType.DMA,
            pltpu.SemaphoreType.DMA,
            pltpu.SemaphoreType.DMA,
            pltpu.SemaphoreType.DMA,
            pltpu.SemaphoreType.DMA,
            pltpu.SemaphoreType.DMA,
        ],
    )
    def k(tab, xf, out, xfb, g0, g1, g2, g3,
          sg0, sg1, sg2, sg3, so0, so1, so2, so3):
        w = lax.axis_index("subcore") * 2 + lax.axis_index("core")
        gbufs = (g0, g1, g2, g3)
        sgs = (sg0, sg1, sg2, sg3)
        sos = (so0, so1, so2, so3)
        base = w * PER_W

        pltpu.sync_copy(xf.at[pl.ds(base, PER_W)], xfb)

        def gather_copy(u, b):
            return pltpu.make_async_copy(
                tab.at[xfb.at[pl.ds(u * UNIT, UNIT)]], gbufs[b], sgs[b])

        def write_copy(u, b):
            return pltpu.make_async_copy(
                gbufs[b], out.at[pl.ds(base + u * UNIT, UNIT), :], sos[b])

        gather_copy(0, 0).start()
        gather_copy(1, 1).start()

        @pl.loop(0, N_UNITS, step=NSLOT)
        def _(u0):
            for db in range(NSLOT):
                u = u0 + db
                b = db % NSLOT
                nb = (db + 2) % NSLOT

                @pl.when(u >= 2)
                def _():
                    write_copy(u - 2, nb).wait()

                @pl.when(u + 2 < N_UNITS)
                def _():
                    gather_copy(u + 2, nb).start()

                gather_copy(u, b).wait()
                write_copy(u, b).start()

        write_copy(N_UNITS - 2, (N_UNITS - 2) % NSLOT).wait()
        write_copy(N_UNITS - 1, (N_UNITS - 1) % NSLOT).wait()

    return k(table, xfl)


def kernel(x, table):
    xT = x.T.astype(jnp.int32)
    xfl = _flatten_idx(xT)
    out = _gather(table, xfl)
    return out.reshape(B0, B1, D)
